# baseline (speedup 1.0000x reference)
import functools

import jax
import jax.numpy as jnp
from jax.experimental import pallas as pl
from jax.experimental.pallas import tpu as pltpu

V = 1000000
D = 100
BLK = 8000
NSTEP = V // BLK


def _tc_reduce(weight, ones):
    def body(w_ref, o_ref, out_ref, acc_ref):
        i = pl.program_id(0)

        @pl.when(i == 0)
        def _():
            acc_ref[...] = jnp.zeros_like(acc_ref)

        acc_ref[...] += jax.lax.dot_general(
            o_ref[...], w_ref[...], (((0,), (0,)), ((), ())),
            preferred_element_type=jnp.float32)

        @pl.when(i == NSTEP - 1)
        def _():
            out_ref[...] = acc_ref[...]

    return pl.pallas_call(
        body,
        grid=(NSTEP,),
        in_specs=[
            pl.BlockSpec((BLK, D), lambda i: (i, 0)),
            pl.BlockSpec((BLK, 1), lambda i: (i, 0)),
        ],
        out_specs=pl.BlockSpec((1, D), lambda i: (0, 0)),
        out_shape=jax.ShapeDtypeStruct((1, D), jnp.float32),
        scratch_shapes=[pltpu.VMEM((1, D), jnp.float32)],
    )(weight, ones)


def kernel(input, weight):
    ones = jnp.ones((V, 1), jnp.float32)
    return _tc_reduce(weight, ones)[0]


# speedup vs baseline: 1.9317x; 1.9317x over previous
import functools

import jax
import jax.numpy as jnp
from jax import lax
from jax.experimental import pallas as pl
from jax.experimental.pallas import tpu as pltpu
from jax.experimental.pallas import tpu_sc as plsc

D = 100
LANES = 16
COL_OFFS = (0, 16, 32, 48, 64, 80, 84)
NACC = len(COL_OFFS)
ACC_W = NACC * LANES

NC = 2
NS = 16
NW = NC * NS

GRP = 64


def _sc_embed_sum(input_idx, weight):
    B = input_idx.shape[0]
    BPW = B // NW
    NGRP = BPW // GRP

    mesh = plsc.VectorSubcoreMesh(core_axis_name="c", subcore_axis_name="s")

    @functools.partial(
        pl.kernel,
        out_type=jax.ShapeDtypeStruct((NW, ACC_W), jnp.float32),
        mesh=mesh,
        scratch_types=[
            pltpu.VMEM((BPW,), jnp.int32),
            pltpu.VMEM((GRP, D), jnp.float32),
            pltpu.VMEM((ACC_W,), jnp.float32),
            pltpu.SemaphoreType.DMA,
            pltpu.SemaphoreType.DMA,
            pltpu.SemaphoreType.DMA,
            pltpu.SemaphoreType.DMA,
        ],
    )
    def k(idx_hbm, tbl_hbm, out_hbm, idx_v, rows_v, acc_v, *sems):
        cid = lax.axis_index("c")
        sid = lax.axis_index("s")
        wid = sid * NC + cid
        base = wid * BPW

        pltpu.sync_copy(idx_hbm.at[pl.ds(base, BPW)], idx_v)

        def body(g, accs):
            v = idx_v[pl.ds(g * GRP, GRP)]
            for lane in range(GRP):
                pltpu.async_copy(tbl_hbm.at[v[lane]], rows_v.at[lane],
                                 sems[lane % 4])
            for q in range(4):
                pltpu.make_async_copy(
                    tbl_hbm.at[pl.ds(0, GRP // 4)],
                    rows_v.at[pl.ds(q * (GRP // 4), GRP // 4)],
                    sems[q]).wait()
            for lane in range(GRP):
                accs = tuple(
                    accs[i] + rows_v[lane, pl.ds(COL_OFFS[i], LANES)]
                    for i in range(NACC)
                )
            return accs

        zero = jnp.zeros((LANES,), jnp.float32)
        accs = lax.fori_loop(0, NGRP, body, (zero,) * NACC)

        for i in range(NACC):
            acc_v[pl.ds(i * LANES, LANES)] = accs[i]
        pltpu.sync_copy(acc_v, out_hbm.at[wid])

    return k(input_idx, weight)


def kernel(input, weight):
    part = _sc_embed_sum(input.astype(jnp.int32), weight)
    w = part.sum(axis=0)
    return jnp.concatenate([w[:96], w[108:112]])
